# baseline (speedup 1.0000x reference)
import math

import jax
import jax.numpy as jnp
from jax import lax
from jax.experimental import pallas as pl
from jax.experimental.pallas import tpu as pltpu
from jax.experimental.pallas import tpu_sc as plsc

D = 64
SCALE = math.sqrt(D)

_NC = 2
_NS = 16
_NW = _NC * _NS
_L = 16


def _iota16():
    return lax.iota(jnp.int32, _L)


def _transpose_slab(slab_v, out_v, npairs):
    for j in range(8):
        dvec = jnp.int32(16 * (j % 4)) + _iota16()
        qbit = j // 4

        @pl.loop(0, npairs, unroll=4)
        def _row(r):
            bvec = jnp.full((_L,), 2 * r + qbit, dtype=jnp.int32)
            vals = plsc.load_gather(slab_v, [dvec, bvec])
            out_v[r, pl.ds(16 * j, _L)] = vals * SCALE


def _make_pack(V: int):
    ncols = V // 128
    rem = V - ncols * 128
    nk = ncols // _NW
    extra = ncols - nk * _NW
    assert nk % 2 == 0, ncols
    mesh = plsc.VectorSubcoreMesh(core_axis_name="c", subcore_axis_name="s")

    def body(tt_hbm, tail_hbm, out_hbm, slab0, slab1, pack0, pack1,
             isem0, isem1, osem0, osem1):
        wid = lax.axis_index("s") * _NC + lax.axis_index("c")
        bufs = ((slab0, pack0, isem0, osem0), (slab1, pack1, isem1, osem1))

        def col(k):
            return wid + k * _NW

        def fire_in(c, buf):
            pltpu.async_copy(tt_hbm.at[:, pl.ds(c * 128, 128)], buf[0],
                             buf[2])

        def wait_in(buf):
            pltpu.make_async_copy(tt_hbm.at[:, pl.ds(0, 128)], buf[0],
                                  buf[2]).wait()

        def fire_out(c, buf):
            pltpu.async_copy(buf[1], out_hbm.at[pl.ds(c * 64, 64)], buf[3])

        def wait_out(buf):
            pltpu.make_async_copy(buf[1], out_hbm.at[pl.ds(0, 64)],
                                  buf[3]).wait()

        fire_in(col(0), bufs[0])
        fire_in(col(1), bufs[1])

        @pl.loop(0, nk, step=2)
        def _main(k):
            for par in range(2):
                kk = k + par
                buf = bufs[par]
                wait_in(buf)

                @pl.when(kk >= 2)
                def _():
                    wait_out(buf)

                _transpose_slab(buf[0], buf[1], 64)
                fire_out(col(kk), buf)

                @pl.when(kk + 2 < nk)
                def _():
                    fire_in(col(kk + 2), buf)

        wait_out(bufs[0])
        wait_out(bufs[1])

        if extra:
            @pl.when(wid < jnp.int32(extra))
            def _():
                c = wid + nk * _NW
                pltpu.sync_copy(tt_hbm.at[:, pl.ds(c * 128, 128)], slab0)
                _transpose_slab(slab0, pack0, 64)
                pltpu.sync_copy(pack0, out_hbm.at[pl.ds(c * 64, 64)])

        if rem:
            @pl.when(wid == jnp.int32(_NW - 1))
            def _():
                pltpu.sync_copy(tail_hbm, slab1.at[pl.ds(0, rem // 2)])

                @pl.loop(0, rem // 2, unroll=2)
                def _r(r):
                    for j in range(8):
                        sl = pl.ds(16 * j, _L)
                        pack1[r, sl] = slab1[r, sl] * SCALE

                pltpu.sync_copy(pack1.at[pl.ds(0, rem // 2)],
                                out_hbm.at[pl.ds(ncols * 64, rem // 2)])

    return pl.kernel(
        body,
        out_type=jax.ShapeDtypeStruct((V // 2, 128), jnp.float32),
        mesh=mesh,
        scratch_types=[
            pltpu.VMEM((D, 128), jnp.float32),
            pltpu.VMEM((D, 128), jnp.float32),
            pltpu.VMEM((64, 128), jnp.float32),
            pltpu.VMEM((64, 128), jnp.float32),
            pltpu.SemaphoreType.DMA,
            pltpu.SemaphoreType.DMA,
            pltpu.SemaphoreType.DMA,
            pltpu.SemaphoreType.DMA,
        ],
        compiler_params=pltpu.CompilerParams(needs_layout_passes=False),
    )


def _make_lookup(B: int, H: int):
    nblk = B // 128
    assert nblk % _NW == 0 and H % 2 == 0, (B, H)
    bpw = nblk // _NW
    mesh = plsc.VectorSubcoreMesh(core_axis_name="c", subcore_axis_name="s")

    def body(pk_hbm, xT_hbm, out_hbm, idx_v, pidx0, pidx1, qc0, qc1,
             rows0, rows1, tr0, tr1, gsem0, gsem1, ssem0, ssem1):
        wid = lax.axis_index("s") * _NC + lax.axis_index("c")
        bufs = ((pidx0, qc0, rows0, tr0, gsem0, ssem0),
                (pidx1, qc1, rows1, tr1, gsem1, ssem1))

        def prep(h, buf):
            pidx, qc = buf[0], buf[1]
            for k in range(8):
                sl = pl.ds(16 * k, _L)
                iv = idx_v[h, sl]
                pidx[sl] = lax.shift_right_logical(iv, 1)
                qc[sl] = (iv & 1) * 64

        def fire_gather(buf):
            pltpu.async_copy(pk_hbm.at[buf[0]], buf[2], buf[4])

        def wait_gather(buf):
            pltpu.make_async_copy(pk_hbm.at[pl.ds(0, 128)], buf[2],
                                  buf[4]).wait()

        def transpose(buf):
            qc, rows, tr = buf[1], buf[2], buf[3]
            for k in range(8):
                rvec = jnp.int32(16 * k) + _iota16()
                qck = qc[pl.ds(16 * k, _L)]

                @pl.loop(0, D, unroll=8)
                def _d(d):
                    vals = plsc.load_gather(rows, [rvec, qck + d])
                    tr[d, pl.ds(16 * k, _L)] = vals

        def fire_store(h, b0, buf):
            pltpu.async_copy(buf[3], out_hbm.at[h, :, pl.ds(b0, 128)],
                             buf[5])

        def wait_store(buf):
            pltpu.make_async_copy(buf[3], out_hbm.at[0, :, pl.ds(0, 128)],
                                  buf[5]).wait()

        @pl.loop(0, bpw)
        def _blk(blk):
            b0 = (wid * bpw + blk) * 128
            pltpu.sync_copy(xT_hbm.at[:, pl.ds(b0, 128)], idx_v)

            prep(0, bufs[0])
            fire_gather(bufs[0])
            prep(1, bufs[1])
            fire_gather(bufs[1])

            @pl.loop(0, H, step=2)
            def _h(h):
                for par in range(2):
                    hh = h + par
                    buf = bufs[par]
                    wait_gather(buf)

                    @pl.when(hh >= 2)
                    def _():
                        wait_store(buf)

                    transpose(buf)
                    fire_store(hh, b0, buf)

                    @pl.when(hh + 2 < H)
                    def _():
                        prep(hh + 2, buf)
                        fire_gather(buf)

            wait_store(bufs[0])
            wait_store(bufs[1])

    return pl.kernel(
        body,
        out_type=jax.ShapeDtypeStruct((H, D, B), jnp.float32),
        mesh=mesh,
        scratch_types=[
            pltpu.VMEM((H, 128), jnp.int32),
            pltpu.VMEM((128,), jnp.int32),
            pltpu.VMEM((128,), jnp.int32),
            pltpu.VMEM((128,), jnp.int32),
            pltpu.VMEM((128,), jnp.int32),
            pltpu.VMEM((128, 128), jnp.float32),
            pltpu.VMEM((128, 128), jnp.float32),
            pltpu.VMEM((D, 128), jnp.float32),
            pltpu.VMEM((D, 128), jnp.float32),
            pltpu.SemaphoreType.DMA,
            pltpu.SemaphoreType.DMA,
            pltpu.SemaphoreType.DMA,
            pltpu.SemaphoreType.DMA,
        ],
        compiler_params=pltpu.CompilerParams(needs_layout_passes=False),
    )


def kernel(x, table):
    batch, hist = x.shape
    V = table.shape[0]
    xT = x.T.astype(jnp.int32)
    tableT = table.T
    ncols = V // 128
    tail = (table[ncols * 128:].reshape(-1, 128)
            if V > ncols * 128 else table[:0].reshape(0, 128))
    packed = _make_pack(V)(tableT, tail)
    out3 = _make_lookup(batch, hist)(packed, xT)
    return out3.transpose(2, 0, 1)


# speedup vs baseline: 1.8489x; 1.8489x over previous
import math

import jax
import jax.numpy as jnp
from jax import lax
from jax.experimental import pallas as pl
from jax.experimental.pallas import tpu as pltpu
from jax.experimental.pallas import tpu_sc as plsc

D_MODEL = 64
SCALE = math.sqrt(D_MODEL)

_NC = 2
_NS = 16
_NW = _NC * _NS

_CHUNK = 256
_IDX_PER_STREAM = 128


def _make_embed(B: int):
    assert B % (_NW * _CHUNK * 2) == 0, B
    bpw = B // _NW
    nchunk = bpw // _CHUNK
    ngath = _CHUNK // _IDX_PER_STREAM

    mesh = plsc.VectorSubcoreMesh(core_axis_name="c", subcore_axis_name="s")

    def body(table_hbm, idx_hbm, out_hbm, idx_v, rows0, rows1, st0, st1,
             gsem0, gsem1, ssem0, ssem1):
        wid = lax.axis_index("s") * _NC + lax.axis_index("c")
        base = wid * bpw

        pltpu.sync_copy(idx_hbm.at[pl.ds(base, bpw)], idx_v)

        def fire_gather(g, rows, gsem):
            for j in range(ngath):
                pltpu.async_copy(
                    table_hbm.at[idx_v.at[pl.ds(g * _CHUNK
                                                + j * _IDX_PER_STREAM,
                                                _IDX_PER_STREAM)]],
                    rows.at[pl.ds(j * _IDX_PER_STREAM, _IDX_PER_STREAM)],
                    gsem,
                )

        def wait_gather(rows, gsem):
            pltpu.make_async_copy(out_hbm.at[pl.ds(0, _CHUNK)], rows,
                                  gsem).wait()

        def fire_store(g, st, ssem):
            pltpu.async_copy(st, out_hbm.at[pl.ds(base + g * _CHUNK, _CHUNK)],
                             ssem)

        def wait_store(st, ssem):
            pltpu.make_async_copy(st, out_hbm.at[pl.ds(0, _CHUNK)],
                                  ssem).wait()

        def scale(rows, st):
            @pl.loop(0, _CHUNK, unroll=8)
            def _scale(r):
                for j in range(D_MODEL // 16):
                    sl = pl.ds(j * 16, 16)
                    st[r, sl] = rows[r, sl] * SCALE

        def step(g, rows, st, gsem, ssem, first=False, last=False):
            wait_gather(rows, gsem)
            if not first:
                wait_store(st, ssem)
            scale(rows, st)
            fire_store(g, st, ssem)
            if not last:
                fire_gather(g + 2, rows, gsem)

        bufs = ((rows0, st0, gsem0, ssem0), (rows1, st1, gsem1, ssem1))

        fire_gather(0, rows0, gsem0)
        fire_gather(1, rows1, gsem1)
        step(0, *bufs[0], first=True)
        step(1, *bufs[1], first=True)

        @pl.loop(2, nchunk - 2, step=2)
        def _steady(g0):
            step(g0, *bufs[0])
            step(g0 + 1, *bufs[1])

        step(nchunk - 2, *bufs[0], last=True)
        step(nchunk - 1, *bufs[1], last=True)
        wait_store(st0, ssem0)
        wait_store(st1, ssem1)

    return pl.kernel(
        body,
        out_type=jax.ShapeDtypeStruct((B, D_MODEL), jnp.float32),
        mesh=mesh,
        scratch_types=[
            pltpu.VMEM((B // _NW,), jnp.int32),
            pltpu.VMEM((_CHUNK, D_MODEL), jnp.float32),
            pltpu.VMEM((_CHUNK, D_MODEL), jnp.float32),
            pltpu.VMEM((_CHUNK, D_MODEL), jnp.float32),
            pltpu.VMEM((_CHUNK, D_MODEL), jnp.float32),
            pltpu.SemaphoreType.DMA,
            pltpu.SemaphoreType.DMA,
            pltpu.SemaphoreType.DMA,
            pltpu.SemaphoreType.DMA,
        ],
        compiler_params=pltpu.CompilerParams(use_tc_tiling_on_sc=False),
    )


def kernel(x, table):
    batch, hist = x.shape
    idx = x.reshape(-1).astype(jnp.int32)
    out = _make_embed(idx.shape[0])(table, idx)
    return out.reshape(batch, hist, D_MODEL)
